# baseline (speedup 1.0000x reference)
import jax
import jax.numpy as jnp
from jax.experimental import pallas as pl

SEQ = 2048
DIM = 1024
ROW_BLOCK = 2048


def _add_block(resp_ref, pos_ref, out_ref):
    out_ref[...] = resp_ref[...] + pos_ref[...]


def kernel(responses, position_table):
    b, s, d = responses.shape
    flat = responses.reshape(b * s, d)
    blocks_per_seq = s // ROW_BLOCK
    out = pl.pallas_call(
        _add_block,
        grid=(blocks_per_seq, b),
        in_specs=[
            pl.BlockSpec((ROW_BLOCK, d), lambda i, j: (j * blocks_per_seq + i, 0)),
            pl.BlockSpec((ROW_BLOCK, d), lambda i, j: (i, 0)),
        ],
        out_specs=pl.BlockSpec((ROW_BLOCK, d), lambda i, j: (j * blocks_per_seq + i, 0)),
        out_shape=jax.ShapeDtypeStruct((b * s, d), responses.dtype),
    )(flat, position_table)
    return out.reshape(b, s, d)
